# baseline (speedup 1.0000x reference)
import functools

import jax
import jax.numpy as jnp
from jax import lax
from jax.experimental import pallas as pl
from jax.experimental.pallas import tpu as pltpu
from jax.experimental.pallas import tpu_sc as plsc

_sds = jax.ShapeDtypeStruct

N = 10000
E = 160000
D = 256
DE = 16

NC = 2
NS = 16
NW = NC * NS

GCH = 128
GCHUNKS = 40
EPW = GCH * GCHUNKS
EPAD = EPW * NW

SCH = 80
EPT = E // NS
SCHUNKS = EPT // SCH
NPAD = 10240
DH = D // 2
ROWS_PT = NPAD // NS

NBLK = 1000
EBLK = 2000


def _silu(x):
    return x * (1.0 / (1.0 + jnp.exp(-x)))


def _pre_body(h_ref, ws_ref, wt_ref, os_ref, ot_ref):
    hb = h_ref[...].astype(jnp.bfloat16)
    os_ref[...] = jnp.dot(hb, ws_ref[...],
                          preferred_element_type=jnp.float32).astype(
                              jnp.bfloat16)
    ot_ref[...] = jnp.dot(hb, wt_ref[...],
                          preferred_element_type=jnp.float32).astype(
                              jnp.bfloat16)


def _edge_body(src_ref, tgt_ref, ea_ref, wee_ref, b1_ref, w2_ref, b2_ref,
               mij_ref):
    x = (src_ref[...].astype(jnp.float32) + tgt_ref[...].astype(jnp.float32)
         + jnp.dot(ea_ref[...].astype(jnp.bfloat16), wee_ref[...],
                   preferred_element_type=jnp.float32)
         + b1_ref[...])
    t = _silu(x)
    y = (jnp.dot(t.astype(jnp.bfloat16), w2_ref[...],
                 preferred_element_type=jnp.float32)
         + b2_ref[...])
    mij_ref[...] = _silu(y)


def _node_body(h_ref, agg_ref, w1h_ref, w1a_ref, b1_ref, w2_ref, b2_ref,
               o_ref):
    hb = h_ref[...]
    x = (jnp.dot(hb.astype(jnp.bfloat16), w1h_ref[...],
                 preferred_element_type=jnp.float32)
         + jnp.dot(agg_ref[...].astype(jnp.bfloat16), w1a_ref[...],
                   preferred_element_type=jnp.float32)
         + b1_ref[...])
    t = _silu(x)
    o_ref[...] = (hb + jnp.dot(t.astype(jnp.bfloat16), w2_ref[...],
                               preferred_element_type=jnp.float32)
                  + b2_ref[...])


def _gather_call(psrc, ptgt, rowp, colp):
    mesh = plsc.VectorSubcoreMesh(core_axis_name="c", subcore_axis_name="s")

    @functools.partial(
        pl.kernel,
        out_type=(_sds((EPAD, DH), jnp.int32), _sds((EPAD, DH), jnp.int32)),
        mesh=mesh,
        scratch_types=[
            pltpu.VMEM((EPW,), jnp.int32),
            pltpu.VMEM((EPW,), jnp.int32),
            pltpu.VMEM((GCH, DH), jnp.int32),
            pltpu.VMEM((GCH, DH), jnp.int32),
            pltpu.VMEM((GCH, DH), jnp.int32),
            pltpu.VMEM((GCH, DH), jnp.int32),
            pltpu.SemaphoreType.DMA,
            pltpu.SemaphoreType.DMA,
            pltpu.SemaphoreType.DMA,
            pltpu.SemaphoreType.DMA,
        ],
    )
    def gather_k(psrc_h, ptgt_h, rowp_h, colp_h, osrc_h, otgt_h,
                 idx_r, idx_c, buf_s0, buf_t0, buf_s1, buf_t1,
                 sem_s0, sem_t0, sem_s1, sem_t1):
        c = lax.axis_index("c")
        s = lax.axis_index("s")
        base = (s * NC + c) * EPW
        pltpu.sync_copy(rowp_h.at[pl.ds(base, EPW)], idx_r)
        pltpu.sync_copy(colp_h.at[pl.ds(base, EPW)], idx_c)
        slots = ((buf_s0, buf_t0, sem_s0, sem_t0),
                 (buf_s1, buf_t1, sem_s1, sem_t1))

        def copies(slot, i):
            buf_s, buf_t, sem_s, sem_t = slot
            ds = pl.ds(i * GCH, GCH)
            return (pltpu.make_async_copy(psrc_h.at[idx_r.at[ds]], buf_s,
                                          sem_s),
                    pltpu.make_async_copy(ptgt_h.at[idx_c.at[ds]], buf_t,
                                          sem_t))

        def start(slot, i):
            for cp in copies(slot, i):
                cp.start()

        def finish(slot, i):
            for cp in copies(slot, i):
                cp.wait()
            buf_s, buf_t, _, _ = slot
            off = base + i * GCH
            pltpu.sync_copy(buf_s, osrc_h.at[pl.ds(off, GCH)])
            pltpu.sync_copy(buf_t, otgt_h.at[pl.ds(off, GCH)])

        start(slots[0], 0)
        start(slots[1], 1)

        def body(k, carry):
            for b in range(2):
                i = 2 * k + b

                @pl.when(i < GCHUNKS)
                def _():
                    finish(slots[b], i)

                @pl.when(i + 2 < GCHUNKS)
                def _():
                    start(slots[b], i + 2)
            return carry

        lax.fori_loop(0, (GCHUNKS + 1) // 2, body, 0)

    return gather_k(psrc, ptgt, rowp, colp)


def _scatter_call(mij, row, zrows):
    mesh = plsc.VectorSubcoreMesh(core_axis_name="c", subcore_axis_name="s")

    @functools.partial(
        pl.kernel,
        out_type=_sds((NPAD, D), jnp.float32),
        mesh=mesh,
        scratch_types=[
            pltpu.VMEM((SCH,), jnp.int32),
            pltpu.VMEM((SCH, DH), jnp.float32),
            pltpu.VMEM_SHARED((NPAD, DH), jnp.float32),
        ],
    )
    def scatter_k(mij_h, row_h, zrows_h, agg_h, idx_v, mbuf, acc):
        c = lax.axis_index("c")
        s = lax.axis_index("s")
        pltpu.sync_copy(zrows_h, acc.at[pl.ds(s * ROWS_PT, ROWS_PT)])
        plsc.subcore_barrier()
        base = s * EPT

        def run_half(col0):
            def chunk(i, carry):
                off = base + i * SCH
                pltpu.sync_copy(row_h.at[pl.ds(off, SCH)], idx_v)
                pltpu.sync_copy(mij_h.at[pl.ds(off, SCH), pl.ds(col0, DH)],
                                mbuf)
                pltpu.sync_copy(mbuf, acc.at[idx_v], add=True)
                return carry

            lax.fori_loop(0, SCHUNKS, chunk, 0)
            plsc.subcore_barrier()
            pltpu.sync_copy(
                acc.at[pl.ds(s * ROWS_PT, ROWS_PT)],
                agg_h.at[pl.ds(s * ROWS_PT, ROWS_PT), pl.ds(col0, DH)])

        @pl.when(c == 0)
        def _():
            run_half(0)

        @pl.when(c == 1)
        def _():
            run_half(DH)

    return scatter_k(mij, row, zrows)


def kernel(h, edge_index, edge_attr, W_e1, b_e1, W_e2, b_e2,
           W_n1, b_n1, W_n2, b_n2):
    f32 = jnp.float32
    row = edge_index[0].astype(jnp.int32)
    col = edge_index[1].astype(jnp.int32)
    pad = jnp.zeros((EPAD - E,), jnp.int32)
    rowp = jnp.concatenate([row, pad])
    colp = jnp.concatenate([col, pad])

    p_src, p_tgt = pl.pallas_call(
        _pre_body,
        grid=(N // NBLK,),
        in_specs=[
            pl.BlockSpec((NBLK, D), lambda i: (i, 0)),
            pl.BlockSpec((D, D), lambda i: (0, 0)),
            pl.BlockSpec((D, D), lambda i: (0, 0)),
        ],
        out_specs=[pl.BlockSpec((NBLK, D), lambda i: (i, 0))] * 2,
        out_shape=[_sds((N, D), jnp.bfloat16)] * 2,
    )(h, W_e1[:D].astype(jnp.bfloat16), W_e1[D:2 * D].astype(jnp.bfloat16))

    def _pack(p):
        return jax.lax.bitcast_convert_type(p.reshape(N, DH, 2), jnp.int32)

    def _unpack(g):
        return jax.lax.bitcast_convert_type(g, jnp.bfloat16).reshape(EPAD, D)

    g_src_i, g_tgt_i = _gather_call(_pack(p_src), _pack(p_tgt), rowp, colp)
    g_src = _unpack(g_src_i)
    g_tgt = _unpack(g_tgt_i)

    mij = pl.pallas_call(
        _edge_body,
        grid=(E // EBLK,),
        in_specs=[
            pl.BlockSpec((EBLK, D), lambda i: (i, 0)),
            pl.BlockSpec((EBLK, D), lambda i: (i, 0)),
            pl.BlockSpec((EBLK, DE), lambda i: (i, 0)),
            pl.BlockSpec((DE, D), lambda i: (0, 0)),
            pl.BlockSpec((1, D), lambda i: (0, 0)),
            pl.BlockSpec((D, D), lambda i: (0, 0)),
            pl.BlockSpec((1, D), lambda i: (0, 0)),
        ],
        out_specs=pl.BlockSpec((EBLK, D), lambda i: (i, 0)),
        out_shape=_sds((E, D), f32),
    )(g_src, g_tgt, edge_attr, W_e1[2 * D:].astype(jnp.bfloat16),
      b_e1.reshape(1, D), W_e2.astype(jnp.bfloat16), b_e2.reshape(1, D))

    zrows = jnp.zeros((ROWS_PT, DH), f32)
    agg = _scatter_call(mij, row, zrows)

    h_out = pl.pallas_call(
        _node_body,
        grid=(N // NBLK,),
        in_specs=[
            pl.BlockSpec((NBLK, D), lambda i: (i, 0)),
            pl.BlockSpec((NBLK, D), lambda i: (i, 0)),
            pl.BlockSpec((D, D), lambda i: (0, 0)),
            pl.BlockSpec((D, D), lambda i: (0, 0)),
            pl.BlockSpec((1, D), lambda i: (0, 0)),
            pl.BlockSpec((D, D), lambda i: (0, 0)),
            pl.BlockSpec((1, D), lambda i: (0, 0)),
        ],
        out_specs=pl.BlockSpec((NBLK, D), lambda i: (i, 0)),
        out_shape=_sds((N, D), f32),
    )(h, agg, W_n1[:D].astype(jnp.bfloat16), W_n1[D:].astype(jnp.bfloat16),
      b_n1.reshape(1, D), W_n2.astype(jnp.bfloat16), b_n2.reshape(1, D))

    return (h_out, mij)


# speedup vs baseline: 2.8672x; 2.8672x over previous
import functools

import jax
import jax.numpy as jnp
from jax import lax
from jax.experimental import pallas as pl
from jax.experimental.pallas import tpu as pltpu
from jax.experimental.pallas import tpu_sc as plsc

_sds = jax.ShapeDtypeStruct

N = 10000
E = 160000
D = 256
DE = 16

NC = 2
NS = 16
NW = NC * NS

GCH = 128
GCHUNKS = 40
EPW = GCH * GCHUNKS
EPAD = EPW * NW

SCH = 80
EPT = E // NS
SCHUNKS = EPT // SCH
NPAD = 10240
DH = D // 2
ROWS_PT = NPAD // NS

NBLK = 1000
EBLK = 2000


def _silu(x):
    return x * (1.0 / (1.0 + jnp.exp(-x)))


def _bf16_bits(x):
    u = jax.lax.bitcast_convert_type(x, jnp.int32)
    r = u + 0x7FFF + ((u >> 16) & 1)
    return (r >> 16) & 0xFFFF


def _pack_halves(p):
    return _bf16_bits(p[:, :DH]) | (_bf16_bits(p[:, DH:]) << 16)


def _unpack_halves(g):
    lo = jax.lax.bitcast_convert_type(g << 16, jnp.float32)
    hi = jax.lax.bitcast_convert_type(g & jnp.int32(-65536), jnp.float32)
    return jnp.concatenate([lo, hi], axis=1)


def _pre_body(h_ref, ws_ref, wt_ref, os_ref, ot_ref):
    hb = h_ref[...].astype(jnp.bfloat16)
    os_ref[...] = _pack_halves(jnp.dot(hb, ws_ref[...],
                                       preferred_element_type=jnp.float32))
    ot_ref[...] = _pack_halves(jnp.dot(hb, wt_ref[...],
                                       preferred_element_type=jnp.float32))


def _edge_body(src_ref, tgt_ref, ea_ref, wee_ref, b1_ref, w2_ref, b2_ref,
               mij_ref):
    x = (_unpack_halves(src_ref[...]) + _unpack_halves(tgt_ref[...])
         + jnp.dot(ea_ref[...].astype(jnp.bfloat16), wee_ref[...],
                   preferred_element_type=jnp.float32)
         + b1_ref[...])
    t = _silu(x)
    y = (jnp.dot(t.astype(jnp.bfloat16), w2_ref[...],
                 preferred_element_type=jnp.float32)
         + b2_ref[...])
    mij_ref[...] = _silu(y)


def _node_body(h_ref, agg_ref, w1h_ref, w1a_ref, b1_ref, w2_ref, b2_ref,
               o_ref):
    hb = h_ref[...]
    x = (jnp.dot(hb.astype(jnp.bfloat16), w1h_ref[...],
                 preferred_element_type=jnp.float32)
         + jnp.dot(agg_ref[...].astype(jnp.bfloat16), w1a_ref[...],
                   preferred_element_type=jnp.float32)
         + b1_ref[...])
    t = _silu(x)
    o_ref[...] = (hb + jnp.dot(t.astype(jnp.bfloat16), w2_ref[...],
                               preferred_element_type=jnp.float32)
                  + b2_ref[...])


def _gather_call(psrc, ptgt, rowp, colp):
    mesh = plsc.VectorSubcoreMesh(core_axis_name="c", subcore_axis_name="s")

    @functools.partial(
        pl.kernel,
        out_type=(_sds((EPAD, DH), jnp.int32), _sds((EPAD, DH), jnp.int32)),
        mesh=mesh,
        scratch_types=[
            pltpu.VMEM((EPW,), jnp.int32),
            pltpu.VMEM((EPW,), jnp.int32),
            pltpu.VMEM((GCH, DH), jnp.int32),
            pltpu.VMEM((GCH, DH), jnp.int32),
            pltpu.VMEM((GCH, DH), jnp.int32),
            pltpu.VMEM((GCH, DH), jnp.int32),
            pltpu.SemaphoreType.DMA,
            pltpu.SemaphoreType.DMA,
            pltpu.SemaphoreType.DMA,
            pltpu.SemaphoreType.DMA,
        ],
    )
    def gather_k(psrc_h, ptgt_h, rowp_h, colp_h, osrc_h, otgt_h,
                 idx_r, idx_c, buf_s0, buf_t0, buf_s1, buf_t1,
                 sem_s0, sem_t0, sem_s1, sem_t1):
        c = lax.axis_index("c")
        s = lax.axis_index("s")
        base = (s * NC + c) * EPW
        pltpu.sync_copy(rowp_h.at[pl.ds(base, EPW)], idx_r)
        pltpu.sync_copy(colp_h.at[pl.ds(base, EPW)], idx_c)
        slots = ((buf_s0, buf_t0, sem_s0, sem_t0),
                 (buf_s1, buf_t1, sem_s1, sem_t1))

        def copies(slot, i):
            buf_s, buf_t, sem_s, sem_t = slot
            ds = pl.ds(i * GCH, GCH)
            return (pltpu.make_async_copy(psrc_h.at[idx_r.at[ds]], buf_s,
                                          sem_s),
                    pltpu.make_async_copy(ptgt_h.at[idx_c.at[ds]], buf_t,
                                          sem_t))

        def start(slot, i):
            for cp in copies(slot, i):
                cp.start()

        def finish(slot, i):
            for cp in copies(slot, i):
                cp.wait()
            buf_s, buf_t, _, _ = slot
            off = base + i * GCH
            pltpu.sync_copy(buf_s, osrc_h.at[pl.ds(off, GCH)])
            pltpu.sync_copy(buf_t, otgt_h.at[pl.ds(off, GCH)])

        start(slots[0], 0)
        start(slots[1], 1)

        def body(k, carry):
            for b in range(2):
                i = 2 * k + b

                @pl.when(i < GCHUNKS)
                def _():
                    finish(slots[b], i)

                @pl.when(i + 2 < GCHUNKS)
                def _():
                    start(slots[b], i + 2)
            return carry

        lax.fori_loop(0, (GCHUNKS + 1) // 2, body, 0)

    return gather_k(psrc, ptgt, rowp, colp)


def _scatter_call(mij, row, zrows):
    mesh = plsc.VectorSubcoreMesh(core_axis_name="c", subcore_axis_name="s")

    @functools.partial(
        pl.kernel,
        out_type=_sds((NPAD, D), jnp.float32),
        mesh=mesh,
        scratch_types=[
            pltpu.VMEM((SCH,), jnp.int32),
            pltpu.VMEM((SCH, DH), jnp.float32),
            pltpu.VMEM_SHARED((NPAD, DH), jnp.float32),
        ],
    )
    def scatter_k(mij_h, row_h, zrows_h, agg_h, idx_v, mbuf, acc):
        c = lax.axis_index("c")
        s = lax.axis_index("s")
        pltpu.sync_copy(zrows_h, acc.at[pl.ds(s * ROWS_PT, ROWS_PT)])
        plsc.subcore_barrier()
        base = s * EPT

        def run_half(col0):
            def chunk(i, carry):
                off = base + i * SCH
                pltpu.sync_copy(row_h.at[pl.ds(off, SCH)], idx_v)
                pltpu.sync_copy(mij_h.at[pl.ds(off, SCH), pl.ds(col0, DH)],
                                mbuf)
                pltpu.sync_copy(mbuf, acc.at[idx_v], add=True)
                return carry

            lax.fori_loop(0, SCHUNKS, chunk, 0)
            plsc.subcore_barrier()
            pltpu.sync_copy(
                acc.at[pl.ds(s * ROWS_PT, ROWS_PT)],
                agg_h.at[pl.ds(s * ROWS_PT, ROWS_PT), pl.ds(col0, DH)])

        @pl.when(c == 0)
        def _():
            run_half(0)

        @pl.when(c == 1)
        def _():
            run_half(DH)

    return scatter_k(mij, row, zrows)


def kernel(h, edge_index, edge_attr, W_e1, b_e1, W_e2, b_e2,
           W_n1, b_n1, W_n2, b_n2):
    f32 = jnp.float32
    row = edge_index[0].astype(jnp.int32)
    col = edge_index[1].astype(jnp.int32)
    pad = jnp.zeros((EPAD - E,), jnp.int32)
    rowp = jnp.concatenate([row, pad])
    colp = jnp.concatenate([col, pad])

    p_src, p_tgt = pl.pallas_call(
        _pre_body,
        grid=(N // NBLK,),
        in_specs=[
            pl.BlockSpec((NBLK, D), lambda i: (i, 0)),
            pl.BlockSpec((D, D), lambda i: (0, 0)),
            pl.BlockSpec((D, D), lambda i: (0, 0)),
        ],
        out_specs=[pl.BlockSpec((NBLK, DH), lambda i: (i, 0))] * 2,
        out_shape=[_sds((N, DH), jnp.int32)] * 2,
    )(h, W_e1[:D].astype(jnp.bfloat16), W_e1[D:2 * D].astype(jnp.bfloat16))

    g_src, g_tgt = _gather_call(p_src, p_tgt, rowp, colp)

    mij = pl.pallas_call(
        _edge_body,
        grid=(E // EBLK,),
        in_specs=[
            pl.BlockSpec((EBLK, DH), lambda i: (i, 0)),
            pl.BlockSpec((EBLK, DH), lambda i: (i, 0)),
            pl.BlockSpec((EBLK, DE), lambda i: (i, 0)),
            pl.BlockSpec((DE, D), lambda i: (0, 0)),
            pl.BlockSpec((1, D), lambda i: (0, 0)),
            pl.BlockSpec((D, D), lambda i: (0, 0)),
            pl.BlockSpec((1, D), lambda i: (0, 0)),
        ],
        out_specs=pl.BlockSpec((EBLK, D), lambda i: (i, 0)),
        out_shape=_sds((E, D), f32),
    )(g_src, g_tgt, edge_attr, W_e1[2 * D:].astype(jnp.bfloat16),
      b_e1.reshape(1, D), W_e2.astype(jnp.bfloat16), b_e2.reshape(1, D))

    zrows = jnp.zeros((ROWS_PT, DH), f32)
    agg = _scatter_call(mij, row, zrows)

    h_out = pl.pallas_call(
        _node_body,
        grid=(N // NBLK,),
        in_specs=[
            pl.BlockSpec((NBLK, D), lambda i: (i, 0)),
            pl.BlockSpec((NBLK, D), lambda i: (i, 0)),
            pl.BlockSpec((D, D), lambda i: (0, 0)),
            pl.BlockSpec((D, D), lambda i: (0, 0)),
            pl.BlockSpec((1, D), lambda i: (0, 0)),
            pl.BlockSpec((D, D), lambda i: (0, 0)),
            pl.BlockSpec((1, D), lambda i: (0, 0)),
        ],
        out_specs=pl.BlockSpec((NBLK, D), lambda i: (i, 0)),
        out_shape=_sds((N, D), f32),
    )(h, agg, W_n1[:D].astype(jnp.bfloat16), W_n1[D:].astype(jnp.bfloat16),
      b_n1.reshape(1, D), W_n2.astype(jnp.bfloat16), b_n2.reshape(1, D))

    return (h_out, mij)


# speedup vs baseline: 3.4392x; 1.1995x over previous
import functools

import jax
import jax.numpy as jnp
from jax import lax
from jax.experimental import pallas as pl
from jax.experimental.pallas import tpu as pltpu
from jax.experimental.pallas import tpu_sc as plsc

_sds = jax.ShapeDtypeStruct

N = 10000
E = 160000
D = 256
DE = 16

NC = 2
NS = 16
NW = NC * NS

GCH = 128
GCHUNKS = 40
EPW = GCH * GCHUNKS
EPAD = EPW * NW

SCH = 80
EPT = E // NS
SCHUNKS = EPT // SCH
NPAD = 10240
DH = D // 2
ROWS_PT = NPAD // NS

NBLK = 1000
EBLK = 2000


def _silu(x):
    return x * (1.0 / (1.0 + jnp.exp(-x)))


def _bf16_bits(x):
    u = jax.lax.bitcast_convert_type(x, jnp.int32)
    r = u + 0x7FFF + ((u >> 16) & 1)
    return (r >> 16) & 0xFFFF


def _pack_halves(p):
    return _bf16_bits(p[:, :DH]) | (_bf16_bits(p[:, DH:]) << 16)


def _unpack_halves(g):
    lo = jax.lax.bitcast_convert_type(g << 16, jnp.float32)
    hi = jax.lax.bitcast_convert_type(g & jnp.int32(-65536), jnp.float32)
    return jnp.concatenate([lo, hi], axis=1)


def _pre_body(h_ref, ws_ref, wt_ref, os_ref, ot_ref):
    hb = h_ref[...].astype(jnp.bfloat16)
    os_ref[...] = _pack_halves(jnp.dot(hb, ws_ref[...],
                                       preferred_element_type=jnp.float32))
    ot_ref[...] = _pack_halves(jnp.dot(hb, wt_ref[...],
                                       preferred_element_type=jnp.float32))


def _edge_body(src_ref, tgt_ref, ea_ref, wee_ref, b1_ref, w2_ref, b2_ref,
               mij_ref):
    x = (_unpack_halves(src_ref[...]) + _unpack_halves(tgt_ref[...])
         + jnp.dot(ea_ref[...].astype(jnp.bfloat16), wee_ref[...],
                   preferred_element_type=jnp.float32)
         + b1_ref[...])
    t = _silu(x)
    y = (jnp.dot(t.astype(jnp.bfloat16), w2_ref[...],
                 preferred_element_type=jnp.float32)
         + b2_ref[...])
    mij_ref[...] = _silu(y)


def _node_body(h_ref, agg_ref, w1h_ref, w1a_ref, b1_ref, w2_ref, b2_ref,
               o_ref):
    hb = h_ref[...]
    x = (jnp.dot(hb.astype(jnp.bfloat16), w1h_ref[...],
                 preferred_element_type=jnp.float32)
         + jnp.dot(agg_ref[...].astype(jnp.bfloat16), w1a_ref[...],
                   preferred_element_type=jnp.float32)
         + b1_ref[...])
    t = _silu(x)
    o_ref[...] = (hb + jnp.dot(t.astype(jnp.bfloat16), w2_ref[...],
                               preferred_element_type=jnp.float32)
                  + b2_ref[...])


def _gather_call(psrc, ptgt, rowp, colp):
    mesh = plsc.VectorSubcoreMesh(core_axis_name="c", subcore_axis_name="s")

    @functools.partial(
        pl.kernel,
        out_type=(_sds((EPAD, DH), jnp.int32), _sds((EPAD, DH), jnp.int32)),
        mesh=mesh,
        scratch_types=[
            pltpu.VMEM((EPW,), jnp.int32),
            pltpu.VMEM((EPW,), jnp.int32),
            pltpu.VMEM((GCH, DH), jnp.int32),
            pltpu.VMEM((GCH, DH), jnp.int32),
            pltpu.VMEM((GCH, DH), jnp.int32),
            pltpu.VMEM((GCH, DH), jnp.int32),
            pltpu.SemaphoreType.DMA,
            pltpu.SemaphoreType.DMA,
            pltpu.SemaphoreType.DMA,
            pltpu.SemaphoreType.DMA,
        ],
    )
    def gather_k(psrc_h, ptgt_h, rowp_h, colp_h, osrc_h, otgt_h,
                 idx_r, idx_c, buf_s0, buf_t0, buf_s1, buf_t1,
                 sem_s0, sem_t0, sem_s1, sem_t1):
        c = lax.axis_index("c")
        s = lax.axis_index("s")
        base = (s * NC + c) * EPW
        pltpu.sync_copy(rowp_h.at[pl.ds(base, EPW)], idx_r)
        pltpu.sync_copy(colp_h.at[pl.ds(base, EPW)], idx_c)
        slots = ((buf_s0, buf_t0, sem_s0, sem_t0),
                 (buf_s1, buf_t1, sem_s1, sem_t1))

        def copies(slot, i):
            buf_s, buf_t, sem_s, sem_t = slot
            ds = pl.ds(i * GCH, GCH)
            return (pltpu.make_async_copy(psrc_h.at[idx_r.at[ds]], buf_s,
                                          sem_s),
                    pltpu.make_async_copy(ptgt_h.at[idx_c.at[ds]], buf_t,
                                          sem_t))

        def start(slot, i):
            for cp in copies(slot, i):
                cp.start()

        def finish(slot, i):
            for cp in copies(slot, i):
                cp.wait()
            buf_s, buf_t, _, _ = slot
            off = base + i * GCH
            pltpu.sync_copy(buf_s, osrc_h.at[pl.ds(off, GCH)])
            pltpu.sync_copy(buf_t, otgt_h.at[pl.ds(off, GCH)])

        start(slots[0], 0)
        start(slots[1], 1)

        def body(k, carry):
            for b in range(2):
                i = 2 * k + b

                @pl.when(i < GCHUNKS)
                def _():
                    finish(slots[b], i)

                @pl.when(i + 2 < GCHUNKS)
                def _():
                    start(slots[b], i + 2)
            return carry

        lax.fori_loop(0, (GCHUNKS + 1) // 2, body, 0)

    return gather_k(psrc, ptgt, rowp, colp)


def _scatter_call(mij, row, zrows):
    mesh = plsc.VectorSubcoreMesh(core_axis_name="c", subcore_axis_name="s")

    @functools.partial(
        pl.kernel,
        out_type=_sds((NPAD, D), jnp.float32),
        mesh=mesh,
        scratch_types=[
            pltpu.VMEM((SCH,), jnp.int32),
            pltpu.VMEM((SCH,), jnp.int32),
            pltpu.VMEM((SCH, DH), jnp.float32),
            pltpu.VMEM((SCH, DH), jnp.float32),
            pltpu.VMEM_SHARED((NPAD, DH), jnp.float32),
            pltpu.SemaphoreType.DMA,
            pltpu.SemaphoreType.DMA,
            pltpu.SemaphoreType.DMA,
            pltpu.SemaphoreType.DMA,
            pltpu.SemaphoreType.DMA,
            pltpu.SemaphoreType.DMA,
        ],
    )
    def scatter_k(mij_h, row_h, zrows_h, agg_h, idx0, idx1, mbuf0, mbuf1, acc,
                  six0, six1, sin0, sin1, sadd0, sadd1):
        c = lax.axis_index("c")
        s = lax.axis_index("s")
        pltpu.sync_copy(zrows_h, acc.at[pl.ds(s * ROWS_PT, ROWS_PT)])
        base = s * EPT
        plsc.subcore_barrier()
        slots = ((idx0, mbuf0, six0, sin0, sadd0),
                 (idx1, mbuf1, six1, sin1, sadd1))

        def run_half(col0):
            def idx_cp(slot, i):
                idx, _, six, _, _ = slot
                return pltpu.make_async_copy(
                    row_h.at[pl.ds(base + i * SCH, SCH)], idx, six)

            def load_cp(slot, i):
                _, mbuf, _, sin, _ = slot
                return pltpu.make_async_copy(
                    mij_h.at[pl.ds(base + i * SCH, SCH), pl.ds(col0, DH)],
                    mbuf, sin)

            def add_cp(slot):
                idx, mbuf, _, _, sadd = slot
                return pltpu.make_async_copy(mbuf, acc.at[idx], sadd)

            def start(slot, i):
                idx_cp(slot, i).start()
                load_cp(slot, i).start()

            start(slots[0], 0)
            start(slots[1], 1)

            def body(k, carry):
                for b in range(2):
                    i = 2 * k + b

                    @pl.when(i < SCHUNKS)
                    def _():
                        idx_cp(slots[b], i).wait()
                        load_cp(slots[b], i).wait()
                        idx, mbuf, _, _, sadd = slots[b]
                        pltpu.async_copy(mbuf, acc.at[idx], sadd, add=True)

                    @pl.when(i + 2 < SCHUNKS)
                    def _():
                        add_cp(slots[b]).wait()
                        start(slots[b], i + 2)
                return carry

            lax.fori_loop(0, (SCHUNKS + 1) // 2, body, 0)
            add_cp(slots[(SCHUNKS - 2) % 2]).wait()
            add_cp(slots[(SCHUNKS - 1) % 2]).wait()
            plsc.subcore_barrier()
            pltpu.sync_copy(
                acc.at[pl.ds(s * ROWS_PT, ROWS_PT)],
                agg_h.at[pl.ds(s * ROWS_PT, ROWS_PT), pl.ds(col0, DH)])

        @pl.when(c == 0)
        def _():
            run_half(0)

        @pl.when(c == 1)
        def _():
            run_half(DH)

    return scatter_k(mij, row, zrows)


def kernel(h, edge_index, edge_attr, W_e1, b_e1, W_e2, b_e2,
           W_n1, b_n1, W_n2, b_n2):
    f32 = jnp.float32
    row = edge_index[0].astype(jnp.int32)
    col = edge_index[1].astype(jnp.int32)
    pad = jnp.zeros((EPAD - E,), jnp.int32)
    rowp = jnp.concatenate([row, pad])
    colp = jnp.concatenate([col, pad])

    p_src, p_tgt = pl.pallas_call(
        _pre_body,
        grid=(N // NBLK,),
        in_specs=[
            pl.BlockSpec((NBLK, D), lambda i: (i, 0)),
            pl.BlockSpec((D, D), lambda i: (0, 0)),
            pl.BlockSpec((D, D), lambda i: (0, 0)),
        ],
        out_specs=[pl.BlockSpec((NBLK, DH), lambda i: (i, 0))] * 2,
        out_shape=[_sds((N, DH), jnp.int32)] * 2,
    )(h, W_e1[:D].astype(jnp.bfloat16), W_e1[D:2 * D].astype(jnp.bfloat16))

    g_src, g_tgt = _gather_call(p_src, p_tgt, rowp, colp)

    mij = pl.pallas_call(
        _edge_body,
        grid=(E // EBLK,),
        in_specs=[
            pl.BlockSpec((EBLK, DH), lambda i: (i, 0)),
            pl.BlockSpec((EBLK, DH), lambda i: (i, 0)),
            pl.BlockSpec((EBLK, DE), lambda i: (i, 0)),
            pl.BlockSpec((DE, D), lambda i: (0, 0)),
            pl.BlockSpec((1, D), lambda i: (0, 0)),
            pl.BlockSpec((D, D), lambda i: (0, 0)),
            pl.BlockSpec((1, D), lambda i: (0, 0)),
        ],
        out_specs=pl.BlockSpec((EBLK, D), lambda i: (i, 0)),
        out_shape=_sds((E, D), f32),
    )(g_src, g_tgt, edge_attr, W_e1[2 * D:].astype(jnp.bfloat16),
      b_e1.reshape(1, D), W_e2.astype(jnp.bfloat16), b_e2.reshape(1, D))

    zrows = jnp.zeros((ROWS_PT, DH), f32)
    agg = _scatter_call(mij, row, zrows)

    h_out = pl.pallas_call(
        _node_body,
        grid=(N // NBLK,),
        in_specs=[
            pl.BlockSpec((NBLK, D), lambda i: (i, 0)),
            pl.BlockSpec((NBLK, D), lambda i: (i, 0)),
            pl.BlockSpec((D, D), lambda i: (0, 0)),
            pl.BlockSpec((D, D), lambda i: (0, 0)),
            pl.BlockSpec((1, D), lambda i: (0, 0)),
            pl.BlockSpec((D, D), lambda i: (0, 0)),
            pl.BlockSpec((1, D), lambda i: (0, 0)),
        ],
        out_specs=pl.BlockSpec((NBLK, D), lambda i: (i, 0)),
        out_shape=_sds((N, D), f32),
    )(h, agg, W_n1[:D].astype(jnp.bfloat16), W_n1[D:].astype(jnp.bfloat16),
      b_n1.reshape(1, D), W_n2.astype(jnp.bfloat16), b_n2.reshape(1, D))

    return (h_out, mij)


# speedup vs baseline: 3.5185x; 1.0231x over previous
import functools

import jax
import jax.numpy as jnp
from jax import lax
from jax.experimental import pallas as pl
from jax.experimental.pallas import tpu as pltpu
from jax.experimental.pallas import tpu_sc as plsc

_sds = jax.ShapeDtypeStruct

N = 10000
E = 160000
D = 256
DE = 16

NC = 2
NS = 16
NW = NC * NS

GCH = 128
GCHUNKS = 40
EPW = GCH * GCHUNKS
EPAD = EPW * NW

SCH = 80
EPT = E // NS
SCHUNKS = EPT // SCH
NPAD = 10240
DH = D // 2
ROWS_PT = NPAD // NS

NBLK = 1000
EBLK = 2000


def _silu(x):
    return x * (1.0 / (1.0 + jnp.exp(-x)))


def _bf16_bits(x):
    u = jax.lax.bitcast_convert_type(x, jnp.int32)
    r = u + 0x7FFF + ((u >> 16) & 1)
    return (r >> 16) & 0xFFFF


def _pack_halves(p):
    return _bf16_bits(p[:, :DH]) | (_bf16_bits(p[:, DH:]) << 16)


def _unpack_halves(g):
    lo = jax.lax.bitcast_convert_type(g << 16, jnp.float32)
    hi = jax.lax.bitcast_convert_type(g & jnp.int32(-65536), jnp.float32)
    return jnp.concatenate([lo, hi], axis=1)


def _pre_body(h_ref, ws_ref, wt_ref, os0_ref, ot0_ref, os1_ref, ot1_ref):
    hb = h_ref[...].astype(jnp.bfloat16)
    ps = _pack_halves(jnp.dot(hb, ws_ref[...],
                              preferred_element_type=jnp.float32))
    pt = _pack_halves(jnp.dot(hb, wt_ref[...],
                              preferred_element_type=jnp.float32))
    os0_ref[...] = ps
    os1_ref[...] = ps
    ot0_ref[...] = pt
    ot1_ref[...] = pt


def _edge_body(src_ref, tgt_ref, ea_ref, wee_ref, b1_ref, w2_ref, b2_ref,
               mij_ref):
    x = (_unpack_halves(src_ref[...]) + _unpack_halves(tgt_ref[...])
         + jnp.dot(ea_ref[...].astype(jnp.bfloat16), wee_ref[...],
                   preferred_element_type=jnp.float32)
         + b1_ref[...])
    t = _silu(x)
    y = (jnp.dot(t.astype(jnp.bfloat16), w2_ref[...],
                 preferred_element_type=jnp.float32)
         + b2_ref[...])
    mij_ref[...] = _silu(y)


def _node_body(h_ref, agg_ref, w1h_ref, w1a_ref, b1_ref, w2_ref, b2_ref,
               o_ref):
    hb = h_ref[...]
    x = (jnp.dot(hb.astype(jnp.bfloat16), w1h_ref[...],
                 preferred_element_type=jnp.float32)
         + jnp.dot(agg_ref[...].astype(jnp.bfloat16), w1a_ref[...],
                   preferred_element_type=jnp.float32)
         + b1_ref[...])
    t = _silu(x)
    o_ref[...] = (hb + jnp.dot(t.astype(jnp.bfloat16), w2_ref[...],
                               preferred_element_type=jnp.float32)
                  + b2_ref[...])


def _gather_call(psrc0, ptgt0, psrc1, ptgt1, rowp, colp):
    mesh = plsc.VectorSubcoreMesh(core_axis_name="c", subcore_axis_name="s")

    @functools.partial(
        pl.kernel,
        out_type=(_sds((EPAD, DH), jnp.int32), _sds((EPAD, DH), jnp.int32)),
        mesh=mesh,
        scratch_types=[
            pltpu.VMEM((EPW,), jnp.int32),
            pltpu.VMEM((EPW,), jnp.int32),
            pltpu.VMEM((GCH, DH), jnp.int32),
            pltpu.VMEM((GCH, DH), jnp.int32),
            pltpu.VMEM((GCH, DH), jnp.int32),
            pltpu.VMEM((GCH, DH), jnp.int32),
            pltpu.SemaphoreType.DMA,
            pltpu.SemaphoreType.DMA,
            pltpu.SemaphoreType.DMA,
            pltpu.SemaphoreType.DMA,
        ],
    )
    def gather_k(psrc0_h, ptgt0_h, psrc1_h, ptgt1_h, rowp_h, colp_h,
                 osrc_h, otgt_h,
                 idx_r, idx_c, buf_s0, buf_t0, buf_s1, buf_t1,
                 sem_s0, sem_t0, sem_s1, sem_t1):
        c = lax.axis_index("c")
        s = lax.axis_index("s")
        base = (s * NC + c) * EPW
        pltpu.sync_copy(rowp_h.at[pl.ds(base, EPW)], idx_r)
        pltpu.sync_copy(colp_h.at[pl.ds(base, EPW)], idx_c)
        slots = ((buf_s0, buf_t0, sem_s0, sem_t0),
                 (buf_s1, buf_t1, sem_s1, sem_t1))

        def run(psrc_h, ptgt_h):
            def copies(slot, i):
                buf_s, buf_t, sem_s, sem_t = slot
                ds = pl.ds(i * GCH, GCH)
                return (pltpu.make_async_copy(psrc_h.at[idx_r.at[ds]], buf_s,
                                              sem_s),
                        pltpu.make_async_copy(ptgt_h.at[idx_c.at[ds]], buf_t,
                                              sem_t))

            def start(slot, i):
                for cp in copies(slot, i):
                    cp.start()

            def finish(slot, i):
                for cp in copies(slot, i):
                    cp.wait()
                buf_s, buf_t, _, _ = slot
                off = base + i * GCH
                pltpu.sync_copy(buf_s, osrc_h.at[pl.ds(off, GCH)])
                pltpu.sync_copy(buf_t, otgt_h.at[pl.ds(off, GCH)])

            start(slots[0], 0)
            start(slots[1], 1)

            def body(k, carry):
                for b in range(2):
                    i = 2 * k + b

                    @pl.when(i < GCHUNKS)
                    def _():
                        finish(slots[b], i)

                    @pl.when(i + 2 < GCHUNKS)
                    def _():
                        start(slots[b], i + 2)
                return carry

            lax.fori_loop(0, (GCHUNKS + 1) // 2, body, 0)

        @pl.when(c == 0)
        def _():
            run(psrc0_h, ptgt0_h)

        @pl.when(c == 1)
        def _():
            run(psrc1_h, ptgt1_h)

    return gather_k(psrc0, ptgt0, psrc1, ptgt1, rowp, colp)


def _scatter_call(mij, row, zrows):
    mesh = plsc.VectorSubcoreMesh(core_axis_name="c", subcore_axis_name="s")

    @functools.partial(
        pl.kernel,
        out_type=_sds((NPAD, D), jnp.float32),
        mesh=mesh,
        scratch_types=[
            pltpu.VMEM((SCH,), jnp.int32),
            pltpu.VMEM((SCH,), jnp.int32),
            pltpu.VMEM((SCH, DH), jnp.float32),
            pltpu.VMEM((SCH, DH), jnp.float32),
            pltpu.VMEM_SHARED((NPAD, DH), jnp.float32),
            pltpu.SemaphoreType.DMA,
            pltpu.SemaphoreType.DMA,
            pltpu.SemaphoreType.DMA,
            pltpu.SemaphoreType.DMA,
            pltpu.SemaphoreType.DMA,
            pltpu.SemaphoreType.DMA,
        ],
    )
    def scatter_k(mij_h, row_h, zrows_h, agg_h, idx0, idx1, mbuf0, mbuf1, acc,
                  six0, six1, sin0, sin1, sadd0, sadd1):
        c = lax.axis_index("c")
        s = lax.axis_index("s")
        pltpu.sync_copy(zrows_h, acc.at[pl.ds(s * ROWS_PT, ROWS_PT)])
        base = s * EPT
        plsc.subcore_barrier()
        slots = ((idx0, mbuf0, six0, sin0, sadd0),
                 (idx1, mbuf1, six1, sin1, sadd1))

        def run_half(col0):
            def idx_cp(slot, i):
                idx, _, six, _, _ = slot
                return pltpu.make_async_copy(
                    row_h.at[pl.ds(base + i * SCH, SCH)], idx, six)

            def load_cp(slot, i):
                _, mbuf, _, sin, _ = slot
                return pltpu.make_async_copy(
                    mij_h.at[pl.ds(base + i * SCH, SCH), pl.ds(col0, DH)],
                    mbuf, sin)

            def add_cp(slot):
                idx, mbuf, _, _, sadd = slot
                return pltpu.make_async_copy(mbuf, acc.at[idx], sadd)

            def start(slot, i):
                idx_cp(slot, i).start()
                load_cp(slot, i).start()

            start(slots[0], 0)
            start(slots[1], 1)

            def body(k, carry):
                for b in range(2):
                    i = 2 * k + b

                    @pl.when(i < SCHUNKS)
                    def _():
                        idx_cp(slots[b], i).wait()
                        load_cp(slots[b], i).wait()
                        idx, mbuf, _, _, sadd = slots[b]
                        pltpu.async_copy(mbuf, acc.at[idx], sadd, add=True)

                    @pl.when(i + 2 < SCHUNKS)
                    def _():
                        add_cp(slots[b]).wait()
                        start(slots[b], i + 2)
                return carry

            lax.fori_loop(0, (SCHUNKS + 1) // 2, body, 0)
            add_cp(slots[(SCHUNKS - 2) % 2]).wait()
            add_cp(slots[(SCHUNKS - 1) % 2]).wait()
            plsc.subcore_barrier()
            pltpu.sync_copy(
                acc.at[pl.ds(s * ROWS_PT, ROWS_PT)],
                agg_h.at[pl.ds(s * ROWS_PT, ROWS_PT), pl.ds(col0, DH)])

        @pl.when(c == 0)
        def _():
            run_half(0)

        @pl.when(c == 1)
        def _():
            run_half(DH)

    return scatter_k(mij, row, zrows)


def kernel(h, edge_index, edge_attr, W_e1, b_e1, W_e2, b_e2,
           W_n1, b_n1, W_n2, b_n2):
    f32 = jnp.float32
    row = edge_index[0].astype(jnp.int32)
    col = edge_index[1].astype(jnp.int32)
    pad = jnp.zeros((EPAD - E,), jnp.int32)
    rowp = jnp.concatenate([row, pad])
    colp = jnp.concatenate([col, pad])

    p_src0, p_tgt0, p_src1, p_tgt1 = pl.pallas_call(
        _pre_body,
        grid=(N // NBLK,),
        in_specs=[
            pl.BlockSpec((NBLK, D), lambda i: (i, 0)),
            pl.BlockSpec((D, D), lambda i: (0, 0)),
            pl.BlockSpec((D, D), lambda i: (0, 0)),
        ],
        out_specs=[pl.BlockSpec((NBLK, DH), lambda i: (i, 0))] * 4,
        out_shape=[_sds((N, DH), jnp.int32)] * 4,
    )(h, W_e1[:D].astype(jnp.bfloat16), W_e1[D:2 * D].astype(jnp.bfloat16))

    g_src, g_tgt = _gather_call(p_src0, p_tgt0, p_src1, p_tgt1, rowp, colp)

    mij = pl.pallas_call(
        _edge_body,
        grid=(E // EBLK,),
        in_specs=[
            pl.BlockSpec((EBLK, DH), lambda i: (i, 0)),
            pl.BlockSpec((EBLK, DH), lambda i: (i, 0)),
            pl.BlockSpec((EBLK, DE), lambda i: (i, 0)),
            pl.BlockSpec((DE, D), lambda i: (0, 0)),
            pl.BlockSpec((1, D), lambda i: (0, 0)),
            pl.BlockSpec((D, D), lambda i: (0, 0)),
            pl.BlockSpec((1, D), lambda i: (0, 0)),
        ],
        out_specs=pl.BlockSpec((EBLK, D), lambda i: (i, 0)),
        out_shape=_sds((E, D), f32),
    )(g_src, g_tgt, edge_attr, W_e1[2 * D:].astype(jnp.bfloat16),
      b_e1.reshape(1, D), W_e2.astype(jnp.bfloat16), b_e2.reshape(1, D))

    zrows = jnp.zeros((ROWS_PT, DH), f32)
    agg = _scatter_call(mij, row, zrows)

    h_out = pl.pallas_call(
        _node_body,
        grid=(N // NBLK,),
        in_specs=[
            pl.BlockSpec((NBLK, D), lambda i: (i, 0)),
            pl.BlockSpec((NBLK, D), lambda i: (i, 0)),
            pl.BlockSpec((D, D), lambda i: (0, 0)),
            pl.BlockSpec((D, D), lambda i: (0, 0)),
            pl.BlockSpec((1, D), lambda i: (0, 0)),
            pl.BlockSpec((D, D), lambda i: (0, 0)),
            pl.BlockSpec((1, D), lambda i: (0, 0)),
        ],
        out_specs=pl.BlockSpec((NBLK, D), lambda i: (i, 0)),
        out_shape=_sds((N, D), f32),
    )(h, agg, W_n1[:D].astype(jnp.bfloat16), W_n1[D:].astype(jnp.bfloat16),
      b_n1.reshape(1, D), W_n2.astype(jnp.bfloat16), b_n2.reshape(1, D))

    return (h_out, mij)
